# baseline (speedup 1.0000x reference)
import functools

import jax
import jax.numpy as jnp
from jax import lax
from jax.experimental import pallas as pl
from jax.experimental.pallas import tpu as pltpu
from jax.experimental.pallas import tpu_sc as plsc

B = 4096
F = 26
V = 100000
D = 64

NC = 2
NS = 16
NW = NC * NS
LANES = 16

VB = 6144
NCH = 16
NSLAB = F * 8
SPW = NSLAB // NW


def _probe_kernel(xt_hbm, tab_hbm, tail_hbm, out_hbm, shared, sem):
  c = lax.axis_index("c")
  s = lax.axis_index("s")
  w = s * NC + c

  def src(slab, ch):
    f = slab // 8
    tr = slab % 8
    return tab_hbm.at[f, pl.ds(tr * 8, 8), pl.ds(ch * VB, VB)]

  def dst(p):
    return shared.at[s, p]

  n_units = 104

  def unit(u):
    g = w * n_units + u
    slab = g // NCH
    ch = g % NCH
    return slab, ch

  s0, c0 = unit(0)
  pltpu.async_copy(src(s0, c0), dst(0), sem)
  s1, c1 = unit(1)
  pltpu.async_copy(src(s1, c1), dst(1), sem)

  def body(u, _):
    su, cu = unit(u)

    @pl.when(u % 2 == 0)
    def _():
      pltpu.make_async_copy(src(su, cu), dst(0), sem).wait()

      @pl.when(u + 2 < n_units)
      def _():
        s2, c2 = unit(u + 2)
        pltpu.async_copy(src(s2, c2), dst(0), sem)

    @pl.when(u % 2 == 1)
    def _():
      pltpu.make_async_copy(src(su, cu), dst(1), sem).wait()

      @pl.when(u + 2 < n_units)
      def _():
        s2, c2 = unit(u + 2)
        pltpu.async_copy(src(s2, c2), dst(1), sem)

    return 0

  lax.fori_loop(0, n_units, body, 0)


@jax.jit
def _probe(xt, tab_p, tab_tail):
  mesh = plsc.VectorSubcoreMesh(
      core_axis_name="c", subcore_axis_name="s", num_cores=NC, num_subcores=NS
  )
  return pl.kernel(
      _probe_kernel,
      out_type=jax.ShapeDtypeStruct((F, D, B), jnp.float32),
      mesh=mesh,
      scratch_types=[
          pltpu.VMEM_SHARED((NS, 2, 8, VB), jnp.float32),
          pltpu.SemaphoreType.DMA,
      ],
      compiler_params=pltpu.CompilerParams(needs_layout_passes=False),
  )(xt, tab_p, tab_tail)


def kernel(X, tables):
  xt = X.T
  tab_p = tables.transpose(0, 2, 1)
  tab_tail = jnp.pad(tables[:, 99968:, :].transpose(0, 2, 1),
                     ((0, 0), (0, 0), (0, 96)))
  out_p = _probe(xt, tab_p, tab_tail)
  return out_p.transpose(2, 0, 1)


# speedup vs baseline: 1.2519x; 1.2519x over previous
import functools

import jax
import jax.numpy as jnp
from jax import lax
from jax.experimental import pallas as pl
from jax.experimental.pallas import tpu as pltpu
from jax.experimental.pallas import tpu_sc as plsc

B = 4096
F = 26
V = 100000
D = 64

NC = 2
NS = 16
NW = NC * NS
LANES = 16

H0 = 50048
H1 = 49920
VT = 99968
NT = 2 * F


def _gather_half(idx_v, row, ob, lo):
  def body(i, pos):
    v16 = idx_v[pl.ds(i * LANES, LANES)]
    off = v16 - lo
    if lo == 0:
      m = v16 < H0
    else:
      m = v16 >= lo
    g = plsc.load_gather(row, [off], mask=m)
    plsc.store_scatter(ob, [pos], g, mask=m)
    return pos + LANES

  lax.fori_loop(0, B // LANES, body, lax.iota(jnp.int32, LANES), unroll=8)


def _emb2_kernel(xt_hbm, tab_hbm, tail_hbm, out_hbm, idxA, idxB, rowA, rowB,
                 obA, obB, semA, semB, semI, semW):
  w = lax.axis_index("s") * NC + lax.axis_index("c")
  roff = 2 * ((w * (F // 2)) // NW)

  def fire_h0(f, d):
    pltpu.async_copy(tab_hbm.at[f, d, pl.ds(0, H0)], rowA, semA)

  def fire_h1(f, d):
    pltpu.async_copy(tab_hbm.at[f, d, pl.ds(H0, H1)], rowB.at[pl.ds(0, H1)],
                     semB)
    pltpu.async_copy(tail_hbm.at[f, d], rowB.at[pl.ds(H1, 128)], semB)

  fire_h0(roff, w)
  fire_h1(roff, w)
  pltpu.async_copy(xt_hbm.at[roff], idxA, semI)

  def half_task(f, d, fn, dn, idx_v, ob, fire_pred=None):
    pltpu.make_async_copy(tab_hbm.at[f, d, pl.ds(0, H0)], rowA, semA).wait()
    _gather_half(idx_v, rowA, ob, 0)

    if fire_pred is None:
      fire_h0(fn, dn)
    else:
      @pl.when(fire_pred)
      def _():
        fire_h0(fn, dn)

    pltpu.make_async_copy(tab_hbm.at[f, d, pl.ds(H0, H1)],
                          rowB.at[pl.ds(0, H1)], semB).wait()
    pltpu.make_async_copy(tail_hbm.at[f, d], rowB.at[pl.ds(H1, 128)],
                          semB).wait()
    _gather_half(idx_v, rowB, ob, H0)

    if fire_pred is None:
      fire_h1(fn, dn)
    else:
      @pl.when(fire_pred)
      def _():
        fire_h1(fn, dn)

    pltpu.async_copy(ob, out_hbm.at[f, d], semW)

  def pair_body(j, _):
    f0 = lax.rem(2 * j + roff, F)
    f1 = f0 + 1
    f0n = lax.rem(f0 + 2, F)

    pltpu.make_async_copy(xt_hbm.at[f0], idxA, semI).wait()
    pltpu.async_copy(xt_hbm.at[f1], idxB, semI)

    @pl.when(j > 0)
    def _():
      pltpu.make_async_copy(obA, out_hbm.at[0, w], semW).wait()
    half_task(f0, w, f0, NW + w, idxA, obA)

    @pl.when(j > 0)
    def _():
      pltpu.make_async_copy(obB, out_hbm.at[0, w], semW).wait()
    half_task(f0, NW + w, f1, w, idxA, obB)

    pltpu.make_async_copy(xt_hbm.at[f1], idxB, semI).wait()

    @pl.when(j + 1 < F // 2)
    def _():
      pltpu.async_copy(xt_hbm.at[f0n], idxA, semI)

    pltpu.make_async_copy(obA, out_hbm.at[0, w], semW).wait()
    half_task(f1, w, f1, NW + w, idxB, obA)

    pltpu.make_async_copy(obB, out_hbm.at[0, w], semW).wait()
    half_task(f1, NW + w, f0n, w, idxB, obB, fire_pred=j + 1 < F // 2)
    return 0

  lax.fori_loop(0, F // 2, pair_body, 0)
  pltpu.make_async_copy(obA, out_hbm.at[0, w], semW).wait()
  pltpu.make_async_copy(obB, out_hbm.at[0, w], semW).wait()


@jax.jit
def _emb2(xt, tab_p, tab_tail):
  mesh = plsc.VectorSubcoreMesh(
      core_axis_name="c", subcore_axis_name="s", num_cores=NC, num_subcores=NS
  )
  return pl.kernel(
      _emb2_kernel,
      out_type=jax.ShapeDtypeStruct((F, D, B), jnp.float32),
      mesh=mesh,
      scratch_types=[
          pltpu.VMEM((B,), jnp.int32),
          pltpu.VMEM((B,), jnp.int32),
          pltpu.VMEM((H0,), jnp.float32),
          pltpu.VMEM((H1 + 128,), jnp.float32),
          pltpu.VMEM((B,), jnp.float32),
          pltpu.VMEM((B,), jnp.float32),
          pltpu.SemaphoreType.DMA,
          pltpu.SemaphoreType.DMA,
          pltpu.SemaphoreType.DMA,
          pltpu.SemaphoreType.DMA,
      ],
      compiler_params=pltpu.CompilerParams(needs_layout_passes=False),
  )(xt, tab_p, tab_tail)


def kernel(X, tables):
  xt = X.T
  tab_p = tables.transpose(0, 2, 1)
  tab_tail = jnp.pad(tables[:, VT:, :].transpose(0, 2, 1), ((0, 0), (0, 0), (0, 96)))
  out_p = _emb2(xt, tab_p, tab_tail)
  return out_p.transpose(2, 0, 1)


# speedup vs baseline: 1.2543x; 1.0019x over previous
import functools

import jax
import jax.numpy as jnp
from jax import lax
from jax.experimental import pallas as pl
from jax.experimental.pallas import tpu as pltpu
from jax.experimental.pallas import tpu_sc as plsc

B = 4096
F = 26
V = 100000
D = 64

NC = 2
NS = 16
NW = NC * NS
LANES = 16

H0 = 50048
H1 = 49920
VT = 99968
NT = 2 * F


def _gather_half(idx_v, row, ob, lo):
  def body(i, pos):
    v16 = idx_v[pl.ds(i * LANES, LANES)]
    off = v16 - lo
    if lo == 0:
      m = v16 < H0
    else:
      m = v16 >= lo
    g = plsc.load_gather(row, [off], mask=m)
    plsc.store_scatter(ob, [pos], g, mask=m)
    return pos + LANES

  lax.fori_loop(0, B // LANES, body, lax.iota(jnp.int32, LANES), unroll=8)


def _emb2_kernel(xt_hbm, tab_hbm, tail_hbm, out_hbm, idxA, idxB, rowA, rowB,
                 obA, obB, semA, semB, semI, semW):
  w = lax.axis_index("s") * NC + lax.axis_index("c")
  roff = lax.rem(2 * ((lax.axis_index("s") * (F // 2)) // NS)
                 + lax.axis_index("c") * 12, F)

  def fire_h0(f, d):
    pltpu.async_copy(tab_hbm.at[f, d, pl.ds(0, H0)], rowA, semA)

  def fire_h1(f, d):
    pltpu.async_copy(tab_hbm.at[f, d, pl.ds(H0, H1)], rowB.at[pl.ds(0, H1)],
                     semB)
    pltpu.async_copy(tail_hbm.at[f, d], rowB.at[pl.ds(H1, 128)], semB)

  fire_h0(roff, w)
  fire_h1(roff, w)
  pltpu.async_copy(xt_hbm.at[roff], idxA, semI)

  def half_task(f, d, fn, dn, idx_v, ob, fire_pred=None):
    pltpu.make_async_copy(tab_hbm.at[f, d, pl.ds(0, H0)], rowA, semA).wait()
    _gather_half(idx_v, rowA, ob, 0)

    if fire_pred is None:
      fire_h0(fn, dn)
    else:
      @pl.when(fire_pred)
      def _():
        fire_h0(fn, dn)

    pltpu.make_async_copy(tab_hbm.at[f, d, pl.ds(H0, H1)],
                          rowB.at[pl.ds(0, H1)], semB).wait()
    pltpu.make_async_copy(tail_hbm.at[f, d], rowB.at[pl.ds(H1, 128)],
                          semB).wait()
    _gather_half(idx_v, rowB, ob, H0)

    if fire_pred is None:
      fire_h1(fn, dn)
    else:
      @pl.when(fire_pred)
      def _():
        fire_h1(fn, dn)

    pltpu.async_copy(ob, out_hbm.at[f, d], semW)

  def pair_body(j, _):
    f0 = lax.rem(2 * j + roff, F)
    f1 = f0 + 1
    f0n = lax.rem(f0 + 2, F)

    pltpu.make_async_copy(xt_hbm.at[f0], idxA, semI).wait()
    pltpu.async_copy(xt_hbm.at[f1], idxB, semI)

    @pl.when(j > 0)
    def _():
      pltpu.make_async_copy(obA, out_hbm.at[0, w], semW).wait()
    half_task(f0, w, f0, NW + w, idxA, obA)

    @pl.when(j > 0)
    def _():
      pltpu.make_async_copy(obB, out_hbm.at[0, w], semW).wait()
    half_task(f0, NW + w, f1, w, idxA, obB)

    pltpu.make_async_copy(xt_hbm.at[f1], idxB, semI).wait()

    @pl.when(j + 1 < F // 2)
    def _():
      pltpu.async_copy(xt_hbm.at[f0n], idxA, semI)

    pltpu.make_async_copy(obA, out_hbm.at[0, w], semW).wait()
    half_task(f1, w, f1, NW + w, idxB, obA)

    pltpu.make_async_copy(obB, out_hbm.at[0, w], semW).wait()
    half_task(f1, NW + w, f0n, w, idxB, obB, fire_pred=j + 1 < F // 2)
    return 0

  lax.fori_loop(0, F // 2, pair_body, 0)
  pltpu.make_async_copy(obA, out_hbm.at[0, w], semW).wait()
  pltpu.make_async_copy(obB, out_hbm.at[0, w], semW).wait()


@jax.jit
def _emb2(xt, tab_p, tab_tail):
  mesh = plsc.VectorSubcoreMesh(
      core_axis_name="c", subcore_axis_name="s", num_cores=NC, num_subcores=NS
  )
  return pl.kernel(
      _emb2_kernel,
      out_type=jax.ShapeDtypeStruct((F, D, B), jnp.float32),
      mesh=mesh,
      scratch_types=[
          pltpu.VMEM((B,), jnp.int32),
          pltpu.VMEM((B,), jnp.int32),
          pltpu.VMEM((H0,), jnp.float32),
          pltpu.VMEM((H1 + 128,), jnp.float32),
          pltpu.VMEM((B,), jnp.float32),
          pltpu.VMEM((B,), jnp.float32),
          pltpu.SemaphoreType.DMA,
          pltpu.SemaphoreType.DMA,
          pltpu.SemaphoreType.DMA,
          pltpu.SemaphoreType.DMA,
      ],
      compiler_params=pltpu.CompilerParams(needs_layout_passes=False),
  )(xt, tab_p, tab_tail)


def kernel(X, tables):
  xt = X.T
  tab_p = tables.transpose(0, 2, 1)
  tab_tail = jnp.pad(tables[:, VT:, :].transpose(0, 2, 1), ((0, 0), (0, 0), (0, 96)))
  out_p = _emb2(xt, tab_p, tab_tail)
  return out_p.transpose(2, 0, 1)
